# baseline (speedup 1.0000x reference)
import jax
import jax.numpy as jnp
from jax import lax
from jax.experimental import pallas as pl
from jax.experimental.pallas import tpu as pltpu
from jax.experimental.pallas import tpu_sc as plsc

B = 16384
A = 200
AP = 201
NC = 2
NS = 16
NW = NC * NS
R = B // NW
L = 16
NCHUNK = 4
CR = R // NCHUNK


def _sc_body(species_hbm, energies_hbm, table_hbm, out_hbm,
             spec0, spec1, en_v, tab_v, res_v, sem0, sem1):
    wid = lax.axis_index("s") * NC + lax.axis_index("c")
    base = wid * R

    pltpu.sync_copy(energies_hbm.at[pl.ds(base, R)], en_v)
    pltpu.sync_copy(table_hbm, tab_v.at[pl.ds(0, 7)])

    bufs = (spec0, spec1)
    sems = (sem0, sem1)

    def start(c):
        return pltpu.async_copy(
            species_hbm.at[pl.ds(base + c * CR, CR), :],
            bufs[c % 2].at[:, pl.ds(0, A)], sems[c % 2])

    lane = lax.iota(jnp.int32, L)
    t_vec = tab_v[...]

    copies = [start(0)]
    for c in range(NCHUNK):
        if c + 1 < NCHUNK:
            copies.append(start(c + 1))
        copies[c].wait()
        buf = bufs[c % 2]

        def row_group(g, _):
            rowv = g * L + lane

            def col_step(j, carry):
                acc0, acc1, c0, c1 = carry
                sv0 = plsc.load_gather(buf, [rowv, c0])
                sv1 = plsc.load_gather(buf, [rowv, c1])
                sae0 = jnp.take_along_axis(t_vec, sv0, axis=0,
                                           mode="promise_in_bounds")
                sae1 = jnp.take_along_axis(t_vec, sv1, axis=0,
                                           mode="promise_in_bounds")
                return acc0 + sae0, acc1 + sae1, c0 + 2, c1 + 2

            z = jnp.zeros((L,), jnp.float32)
            zero = jnp.zeros((L,), jnp.int32)
            one = zero + 1
            acc0, acc1, _, _ = lax.fori_loop(
                0, A // 2, col_step, (z, z, zero, one), unroll=4)
            acc = acc0 + acc1
            rbase = c * CR + g * L
            res_v[pl.ds(rbase, L)] = acc + en_v[pl.ds(rbase, L)]
            return 0

        lax.fori_loop(0, CR // L, row_group, 0)

    pltpu.sync_copy(res_v, out_hbm.at[pl.ds(base, R)])


@jax.jit
def _shift(species, energies, self_energies):
    mesh = plsc.VectorSubcoreMesh(core_axis_name="c", subcore_axis_name="s")
    fn = pl.kernel(
        _sc_body,
        mesh=mesh,
        compiler_params=pltpu.CompilerParams(use_tc_tiling_on_sc=False,
                                             needs_layout_passes=False),
        out_type=jax.ShapeDtypeStruct((B,), jnp.float32),
        scratch_types=[
            pltpu.VMEM((CR, AP), jnp.int32),
            pltpu.VMEM((CR, AP), jnp.int32),
            pltpu.VMEM((R,), jnp.float32),
            pltpu.VMEM((L,), jnp.float32),
            pltpu.VMEM((R,), jnp.float32),
            pltpu.SemaphoreType.DMA,
            pltpu.SemaphoreType.DMA,
        ],
    )
    return fn(species, energies, self_energies)


def kernel(species, energies, self_energies):
    out = _shift(species.astype(jnp.int32), energies, self_energies)
    return (species, out)


# speedup vs baseline: 2.1597x; 2.1597x over previous
import jax
import jax.numpy as jnp
from jax import lax
from jax.experimental import pallas as pl
from jax.experimental.pallas import tpu as pltpu
from jax.experimental.pallas import tpu_sc as plsc

B = 16384
A = 200
NC = 2
NS = 16
NW = NC * NS
R = B // NW
L = 16
TR = B // 128
CHI = A // 8
KPW = R // 128


def _sc_body(spec_hbm, energies_hbm, table_hbm, out_hbm,
             buf0, buf1, en_v, tab_v, res_v, sem0, sem1):
    wid = lax.axis_index("s") * NC + lax.axis_index("c")
    base = wid * R

    pltpu.sync_copy(energies_hbm.at[pl.ds(base, R)], en_v)
    pltpu.sync_copy(table_hbm, tab_v.at[pl.ds(0, 7)])

    bufs = (buf0, buf1)
    sems = (sem0, sem1)

    def start(k):
        return pltpu.async_copy(
            spec_hbm.at[:, wid * KPW + k, :], bufs[k % 2], sems[k % 2])

    t_vec = tab_v[...]

    copies = [start(0)]
    for k in range(KPW):
        if k + 1 < KPW:
            copies.append(start(k + 1))
        copies[k].wait()
        buf = bufs[k % 2]

        def lane_group(g, _):
            g16 = g * L

            def col_tile(chi, carry):
                acc0, acc1 = carry
                for clo in range(8):
                    sv = buf[chi, pl.ds(clo * 128 + g16, L)]
                    sae = jnp.take_along_axis(t_vec, sv, axis=0,
                                              mode="promise_in_bounds")
                    if clo % 2 == 0:
                        acc0 = acc0 + sae
                    else:
                        acc1 = acc1 + sae
                return acc0, acc1

            z = jnp.zeros((L,), jnp.float32)
            acc0, acc1 = lax.fori_loop(0, CHI, col_tile, (z, z))
            rbase = k * 128 + g16
            res_v[pl.ds(rbase, L)] = (acc0 + acc1) + en_v[pl.ds(rbase, L)]
            return 0

        lax.fori_loop(0, 128 // L, lane_group, 0)

    pltpu.sync_copy(res_v, out_hbm.at[pl.ds(base, R)])


@jax.jit
def _shift(spec_lin, energies, self_energies):
    mesh = plsc.VectorSubcoreMesh(core_axis_name="c", subcore_axis_name="s")
    fn = pl.kernel(
        _sc_body,
        mesh=mesh,
        compiler_params=pltpu.CompilerParams(use_tc_tiling_on_sc=False,
                                             needs_layout_passes=False),
        out_type=jax.ShapeDtypeStruct((B,), jnp.float32),
        scratch_types=[
            pltpu.VMEM((CHI, 1024), jnp.int32),
            pltpu.VMEM((CHI, 1024), jnp.int32),
            pltpu.VMEM((R,), jnp.float32),
            pltpu.VMEM((L,), jnp.float32),
            pltpu.VMEM((R,), jnp.float32),
            pltpu.SemaphoreType.DMA,
            pltpu.SemaphoreType.DMA,
        ],
    )
    return fn(spec_lin, energies, self_energies)


def kernel(species, energies, self_energies):
    spec_lin = (species.astype(jnp.int32)
                .reshape(TR, 128, CHI, 8)
                .transpose(2, 0, 3, 1)
                .reshape(CHI, TR, 1024))
    out = _shift(spec_lin, energies, self_energies)
    return (species, out)


# speedup vs baseline: 2.6728x; 1.2376x over previous
import jax
import jax.numpy as jnp
from jax import lax
from jax.experimental import pallas as pl
from jax.experimental.pallas import tpu as pltpu
from jax.experimental.pallas import tpu_sc as plsc

B = 16384
A = 200
NC = 2
NS = 16
NW = NC * NS
R = B // NW
L = 16
TR = B // 128
CHI = A // 8
KPW = R // 128


def _sc_body(spec_hbm, energies_hbm, table_hbm, spec_out_hbm, out_hbm,
             buf0, buf1, buf2, buf3, en_v, tab_v, res_v,
             rs0, rs1, rs2, rs3, wsem):
    wid = lax.axis_index("s") * NC + lax.axis_index("c")
    base = wid * R

    bufs = (buf0, buf1, buf2, buf3)
    rsems = (rs0, rs1, rs2, rs3)

    reads = [
        pltpu.async_copy(spec_hbm.at[:, wid * KPW + k, :], bufs[k], rsems[k])
        for k in range(KPW)
    ]

    pltpu.sync_copy(energies_hbm.at[pl.ds(base, R)], en_v)
    pltpu.sync_copy(table_hbm, tab_v.at[pl.ds(0, 7)])
    t_vec = tab_v[...]

    writes = []
    for k in range(KPW):
        reads[k].wait()
        buf = bufs[k]

        def lane_group(g, _):
            g16 = g * L

            def col_tile(chi, carry):
                acc0, acc1 = carry
                for clo in range(8):
                    sv = buf[chi, pl.ds(clo * 128 + g16, L)]
                    sae = jnp.take_along_axis(t_vec, sv, axis=0,
                                              mode="promise_in_bounds")
                    if clo % 2 == 0:
                        acc0 = acc0 + sae
                    else:
                        acc1 = acc1 + sae
                return acc0, acc1

            z = jnp.zeros((L,), jnp.float32)
            acc0, acc1 = lax.fori_loop(0, CHI, col_tile, (z, z))
            rbase = k * 128 + g16
            res_v[pl.ds(rbase, L)] = (acc0 + acc1) + en_v[pl.ds(rbase, L)]
            return 0

        lax.fori_loop(0, 128 // L, lane_group, 0)
        writes.append(pltpu.async_copy(
            buf, spec_out_hbm.at[:, wid * KPW + k, :], wsem))

    pltpu.sync_copy(res_v, out_hbm.at[pl.ds(base, R)])
    for w in writes:
        w.wait()


@jax.jit
def _shift(spec_lin, energies, self_energies):
    mesh = plsc.VectorSubcoreMesh(core_axis_name="c", subcore_axis_name="s")
    fn = pl.kernel(
        _sc_body,
        mesh=mesh,
        compiler_params=pltpu.CompilerParams(use_tc_tiling_on_sc=False,
                                             needs_layout_passes=False),
        out_type=(
            jax.ShapeDtypeStruct((CHI, TR, 1024), jnp.int32),
            jax.ShapeDtypeStruct((B,), jnp.float32),
        ),
        scratch_types=[
            pltpu.VMEM((CHI, 1024), jnp.int32),
            pltpu.VMEM((CHI, 1024), jnp.int32),
            pltpu.VMEM((CHI, 1024), jnp.int32),
            pltpu.VMEM((CHI, 1024), jnp.int32),
            pltpu.VMEM((R,), jnp.float32),
            pltpu.VMEM((L,), jnp.float32),
            pltpu.VMEM((R,), jnp.float32),
            pltpu.SemaphoreType.DMA,
            pltpu.SemaphoreType.DMA,
            pltpu.SemaphoreType.DMA,
            pltpu.SemaphoreType.DMA,
            pltpu.SemaphoreType.DMA,
        ],
    )
    return fn(spec_lin, energies, self_energies)


def kernel(species, energies, self_energies):
    spec_lin = (species.astype(jnp.int32)
                .reshape(TR, 128, CHI, 8)
                .transpose(2, 0, 3, 1)
                .reshape(CHI, TR, 1024))
    spec_out, out = _shift(spec_lin, energies, self_energies)
    species_out = (spec_out
                   .reshape(CHI, TR, 8, 128)
                   .transpose(1, 3, 0, 2)
                   .reshape(B, A)
                   .astype(species.dtype))
    return (species_out, out)
